# baseline (speedup 1.0000x reference)
import functools

import jax
import jax.numpy as jnp
from jax import lax
from jax.experimental import pallas as pl
from jax.experimental.pallas import tpu as pltpu
from jax.experimental.pallas import tpu_sc as plsc

_N = 50000
_NP = 50048
_D = 64
_H = 32
_E = 800000
_LANES = 16
_MICRO = 128
_SUPER_ROWS = 16
_N_SUPER = 25
_TILES = 16
_NROWBUF = 8
_LOOKAHEAD = 7
_NMSG = 2
_ROWS_PER_TILE = _SUPER_ROWS * _N_SUPER
_E_PAD = _TILES * _ROWS_PER_TILE * _MICRO
_ROWS2D = _E_PAD // _MICRO
_RPT = _NP // _TILES


def _sc_side(table_lo, table_hi, src2, dst2, w2, zeros):
    mesh = plsc.VectorSubcoreMesh(core_axis_name="c", subcore_axis_name="s")

    @functools.partial(
        pl.kernel,
        mesh=mesh,
        out_type=(
            jax.ShapeDtypeStruct((_NP, _H), jnp.float32),
            jax.ShapeDtypeStruct((_NP, _H), jnp.float32),
        ),
        scratch_types=[
            pltpu.VMEM((_SUPER_ROWS, _MICRO), jnp.int32),
            pltpu.VMEM((_SUPER_ROWS, _MICRO), jnp.float32),
            [pltpu.VMEM((_MICRO,), jnp.int32) for _ in range(_SUPER_ROWS)],
            [pltpu.VMEM((_MICRO, _LANES), jnp.int32) for _ in range(_NROWBUF)],
            [pltpu.VMEM((_MICRO, _H), jnp.float32) for _ in range(_NMSG)],
            pltpu.VMEM_SHARED((_NP, _H), jnp.float32),
            pltpu.SemaphoreType.DMA,
            [pltpu.SemaphoreType.DMA for _ in range(_NROWBUF)],
            [pltpu.SemaphoreType.DMA for _ in range(_NMSG)],
        ],
        compiler_params=pltpu.CompilerParams(use_tc_tiling_on_sc=False,
                                             needs_layout_passes=False),
    )
    def k(tlo, thi, src_h, dst_h, w_h, z_h, out_lo, out_hi,
          src_v, w_v, ibufs, rows, msgs, acc, stg_sem, gsems, ssems):
        c = lax.axis_index("c")
        s = lax.axis_index("s")

        def run(table, out):
            pltpu.sync_copy(z_h.at[pl.ds(s * _RPT, _RPT)],
                            acc.at[pl.ds(s * _RPT, _RPT)])
            plsc.subcore_barrier()

            def do_super(v, carry):
                row0 = s * _ROWS_PER_TILE + v * _SUPER_ROWS
                stg = [
                    pltpu.async_copy(src_h.at[pl.ds(row0, _SUPER_ROWS)],
                                     src_v, stg_sem),
                    pltpu.async_copy(w_h.at[pl.ds(row0, _SUPER_ROWS)],
                                     w_v, stg_sem),
                ]
                for j in range(_SUPER_ROWS):
                    stg.append(pltpu.async_copy(dst_h.at[row0 + j],
                                                ibufs[j], stg_sem))
                for d in stg:
                    d.wait()

                gds = {}
                for j in range(_LOOKAHEAD):
                    gds[j] = pltpu.async_copy(table.at[src_v.at[j]],
                                              rows[j % _NROWBUF],
                                              gsems[j % _NROWBUF])
                sds = {}
                for j in range(_SUPER_ROWS):
                    kb = j % _NROWBUF
                    km = j % _NMSG
                    gds[j].wait()
                    if j >= _NMSG:
                        sds[j - _NMSG].wait()
                    rv = rows[kb]
                    mv = msgs[km]

                    def mul_body(g, carry2, _rv=rv, _mv=mv, _j=j):
                        base = pl.multiple_of(g * _LANES, _LANES)
                        wv = w_v[_j, pl.ds(base, _LANES)]
                        for i in range(_LANES):
                            e = base + i
                            ws = wv[i]
                            xi = _rv[e, pl.ds(0, _LANES)]
                            lo = plsc.bitcast(xi << 16, jnp.float32)
                            hi = plsc.bitcast(xi & jnp.int32(-65536),
                                              jnp.float32)
                            _mv[e, pl.ds(0, _LANES)] = lo * ws
                            _mv[e, pl.ds(_LANES, _LANES)] = hi * ws
                        return carry2

                    lax.fori_loop(0, _MICRO // _LANES, mul_body, 0)
                    sds[j] = pltpu.async_copy(mv, acc.at[ibufs[j]],
                                              ssems[km], add=True)
                    jn = j + _LOOKAHEAD
                    if jn < _SUPER_ROWS:
                        gds[jn] = pltpu.async_copy(table.at[src_v.at[jn]],
                                                   rows[jn % _NROWBUF],
                                                   gsems[jn % _NROWBUF])
                for j in range(_SUPER_ROWS - _NMSG, _SUPER_ROWS):
                    sds[j].wait()
                return carry

            lax.fori_loop(0, _N_SUPER, do_super, 0)
            plsc.subcore_barrier()
            pltpu.sync_copy(acc.at[pl.ds(s * _RPT, _RPT)],
                            out.at[pl.ds(s * _RPT, _RPT)])

        @pl.when(c == 0)
        def _():
            run(tlo, out_lo)

        @pl.when(c == 1)
        def _():
            run(thi, out_hi)

    return k(table_lo, table_hi, src2, dst2, w2, zeros)


def _pack_half(a, b):
    ai = lax.bitcast_convert_type(a, jnp.int32) + 32768
    bi = lax.bitcast_convert_type(b, jnp.int32) + 32768
    return lax.shift_right_logical(ai, 16) | (bi & jnp.int32(-65536))


def _tc_dense(side_lo, side_hi, ego, Wg, bg, Wb, bb):
    B = _RPT

    def body(slo, shi, eref, wg, bgr, wb, bbr, oe, olo, ohi):
        side = jnp.concatenate([slo[...], shi[...]], axis=1)
        ego_b = eref[...]
        x = jnp.dot(side, wg[...], preferred_element_type=jnp.float32) + bgr[...]
        sum_emb = jnp.where(x >= 0, x, 0.2 * x)
        y = jnp.dot(ego_b * side, wb[...], preferred_element_type=jnp.float32) + bbr[...]
        bi_emb = jnp.where(y >= 0, y, 0.2 * y)
        o = sum_emb + bi_emb
        nrm = jnp.sqrt(jnp.sum(o * o, axis=1, keepdims=True))
        o = o / (nrm + 1e-12)
        oe[...] = o
        olo[...] = _pack_half(o[:, 0:16], o[:, 16:32])
        ohi[...] = _pack_half(o[:, 32:48], o[:, 48:64])

    f = pl.pallas_call(
        body,
        grid=(_NP // B,),
        in_specs=[
            pl.BlockSpec((B, _H), lambda i: (i, 0)),
            pl.BlockSpec((B, _H), lambda i: (i, 0)),
            pl.BlockSpec((B, _D), lambda i: (i, 0)),
            pl.BlockSpec((_D, _D), lambda i: (0, 0)),
            pl.BlockSpec((1, _D), lambda i: (0, 0)),
            pl.BlockSpec((_D, _D), lambda i: (0, 0)),
            pl.BlockSpec((1, _D), lambda i: (0, 0)),
        ],
        out_specs=[
            pl.BlockSpec((B, _D), lambda i: (i, 0)),
            pl.BlockSpec((B, _LANES), lambda i: (i, 0)),
            pl.BlockSpec((B, _LANES), lambda i: (i, 0)),
        ],
        out_shape=[
            jax.ShapeDtypeStruct((_NP, _D), jnp.float32),
            jax.ShapeDtypeStruct((_NP, _LANES), jnp.int32),
            jax.ShapeDtypeStruct((_NP, _LANES), jnp.int32),
        ],
    )
    return f(side_lo, side_hi, ego, Wg, bg, Wb, bb)


def kernel(user_emb, item_emb,
           W_gc_0, b_gc_0, W_bi_0, b_bi_0,
           W_gc_1, b_gc_1, W_bi_1, b_bi_1,
           W_gc_2, b_gc_2, W_bi_2, b_bi_2,
           edge_weight, edge_index):
    Ws = [(W_gc_0, b_gc_0, W_bi_0, b_bi_0),
          (W_gc_1, b_gc_1, W_bi_1, b_bi_1),
          (W_gc_2, b_gc_2, W_bi_2, b_bi_2)]
    ego = jnp.concatenate([user_emb, item_emb], axis=0)
    ego = jnp.pad(ego, ((0, _NP - _N), (0, 0)))
    src = edge_index[0].astype(jnp.int32)
    dst = edge_index[1].astype(jnp.int32)
    w = edge_weight.astype(jnp.float32)
    pad = _E_PAD - _E
    src2 = jnp.pad(src, (0, pad)).reshape(_ROWS2D, _MICRO)
    dst2 = jnp.pad(dst, (0, pad)).reshape(_ROWS2D, _MICRO)
    w2 = jnp.pad(w, (0, pad)).reshape(_ROWS2D, _MICRO)
    zeros = jnp.zeros((_NP, _H), jnp.float32)
    tlo = _pack_half(ego[:, 0:16], ego[:, 16:32])
    thi = _pack_half(ego[:, 32:48], ego[:, 48:64])
    outs = [ego[:_N]]
    for (Wg, bg, Wb, bb) in Ws:
        slo, shi = _sc_side(tlo, thi, src2, dst2, w2, zeros)
        ego, tlo, thi = _tc_dense(slo, shi, ego, Wg, bg, Wb, bb)
        outs.append(ego[:_N])
    return jnp.concatenate(outs, axis=1)


# speedup vs baseline: 1.0006x; 1.0006x over previous
import functools

import jax
import jax.numpy as jnp
from jax import lax
from jax.experimental import pallas as pl
from jax.experimental.pallas import tpu as pltpu
from jax.experimental.pallas import tpu_sc as plsc

_N = 50000
_NP = 50048
_D = 64
_H = 32
_E = 800000
_LANES = 16
_MICRO = 128
_SUPER_ROWS = 16
_N_SUPER = 25
_TILES = 16
_NROWBUF = 6
_LOOKAHEAD = 5
_NMSG = 3
_ROWS_PER_TILE = _SUPER_ROWS * _N_SUPER
_E_PAD = _TILES * _ROWS_PER_TILE * _MICRO
_ROWS2D = _E_PAD // _MICRO
_RPT = _NP // _TILES


def _sc_side(table_lo, table_hi, src2, dst2, w2, zeros):
    mesh = plsc.VectorSubcoreMesh(core_axis_name="c", subcore_axis_name="s")

    @functools.partial(
        pl.kernel,
        mesh=mesh,
        out_type=(
            jax.ShapeDtypeStruct((_NP, _H), jnp.float32),
            jax.ShapeDtypeStruct((_NP, _H), jnp.float32),
        ),
        scratch_types=[
            pltpu.VMEM((_SUPER_ROWS, _MICRO), jnp.int32),
            pltpu.VMEM((_SUPER_ROWS, _MICRO), jnp.float32),
            [pltpu.VMEM((_MICRO,), jnp.int32) for _ in range(_SUPER_ROWS)],
            [pltpu.VMEM((_MICRO, _LANES), jnp.int32) for _ in range(_NROWBUF)],
            [pltpu.VMEM((_MICRO, _H), jnp.float32) for _ in range(_NMSG)],
            pltpu.VMEM_SHARED((_NP, _H), jnp.float32),
            pltpu.SemaphoreType.DMA,
            [pltpu.SemaphoreType.DMA for _ in range(_NROWBUF)],
            [pltpu.SemaphoreType.DMA for _ in range(_NMSG)],
        ],
        compiler_params=pltpu.CompilerParams(use_tc_tiling_on_sc=False,
                                             needs_layout_passes=False),
    )
    def k(tlo, thi, src_h, dst_h, w_h, z_h, out_lo, out_hi,
          src_v, w_v, ibufs, rows, msgs, acc, stg_sem, gsems, ssems):
        c = lax.axis_index("c")
        s = lax.axis_index("s")

        def run(table, out):
            pltpu.sync_copy(z_h.at[pl.ds(s * _RPT, _RPT)],
                            acc.at[pl.ds(s * _RPT, _RPT)])
            plsc.subcore_barrier()

            def do_super(v, carry):
                row0 = s * _ROWS_PER_TILE + v * _SUPER_ROWS
                stg = [
                    pltpu.async_copy(src_h.at[pl.ds(row0, _SUPER_ROWS)],
                                     src_v, stg_sem),
                    pltpu.async_copy(w_h.at[pl.ds(row0, _SUPER_ROWS)],
                                     w_v, stg_sem),
                ]
                for j in range(_SUPER_ROWS):
                    stg.append(pltpu.async_copy(dst_h.at[row0 + j],
                                                ibufs[j], stg_sem))
                for d in stg:
                    d.wait()

                gds = {}
                for j in range(_LOOKAHEAD):
                    gds[j] = pltpu.async_copy(table.at[src_v.at[j]],
                                              rows[j % _NROWBUF],
                                              gsems[j % _NROWBUF])
                sds = {}
                for j in range(_SUPER_ROWS):
                    kb = j % _NROWBUF
                    km = j % _NMSG
                    gds[j].wait()
                    if j >= _NMSG:
                        sds[j - _NMSG].wait()
                    rv = rows[kb]
                    mv = msgs[km]

                    def mul_body(g, carry2, _rv=rv, _mv=mv, _j=j):
                        base = pl.multiple_of(g * _LANES, _LANES)
                        wv = w_v[_j, pl.ds(base, _LANES)]
                        for i in range(_LANES):
                            e = base + i
                            ws = wv[i]
                            xi = _rv[e, pl.ds(0, _LANES)]
                            lo = plsc.bitcast(xi << 16, jnp.float32)
                            hi = plsc.bitcast(xi & jnp.int32(-65536),
                                              jnp.float32)
                            _mv[e, pl.ds(0, _LANES)] = lo * ws
                            _mv[e, pl.ds(_LANES, _LANES)] = hi * ws
                        return carry2

                    lax.fori_loop(0, _MICRO // _LANES, mul_body, 0)
                    sds[j] = pltpu.async_copy(mv, acc.at[ibufs[j]],
                                              ssems[km], add=True)
                    jn = j + _LOOKAHEAD
                    if jn < _SUPER_ROWS:
                        gds[jn] = pltpu.async_copy(table.at[src_v.at[jn]],
                                                   rows[jn % _NROWBUF],
                                                   gsems[jn % _NROWBUF])
                for j in range(_SUPER_ROWS - _NMSG, _SUPER_ROWS):
                    sds[j].wait()
                return carry

            lax.fori_loop(0, _N_SUPER, do_super, 0)
            plsc.subcore_barrier()
            pltpu.sync_copy(acc.at[pl.ds(s * _RPT, _RPT)],
                            out.at[pl.ds(s * _RPT, _RPT)])

        @pl.when(c == 0)
        def _():
            run(tlo, out_lo)

        @pl.when(c == 1)
        def _():
            run(thi, out_hi)

    return k(table_lo, table_hi, src2, dst2, w2, zeros)


def _pack_half(a, b):
    ai = lax.bitcast_convert_type(a, jnp.int32) + 32768
    bi = lax.bitcast_convert_type(b, jnp.int32) + 32768
    return lax.shift_right_logical(ai, 16) | (bi & jnp.int32(-65536))


def _tc_dense(side_lo, side_hi, ego, Wg, bg, Wb, bb):
    B = _RPT

    def body(slo, shi, eref, wg, bgr, wb, bbr, oe, olo, ohi):
        side = jnp.concatenate([slo[...], shi[...]], axis=1)
        ego_b = eref[...]
        x = jnp.dot(side, wg[...], preferred_element_type=jnp.float32) + bgr[...]
        sum_emb = jnp.where(x >= 0, x, 0.2 * x)
        y = jnp.dot(ego_b * side, wb[...], preferred_element_type=jnp.float32) + bbr[...]
        bi_emb = jnp.where(y >= 0, y, 0.2 * y)
        o = sum_emb + bi_emb
        nrm = jnp.sqrt(jnp.sum(o * o, axis=1, keepdims=True))
        o = o / (nrm + 1e-12)
        oe[...] = o
        olo[...] = _pack_half(o[:, 0:16], o[:, 16:32])
        ohi[...] = _pack_half(o[:, 32:48], o[:, 48:64])

    f = pl.pallas_call(
        body,
        grid=(_NP // B,),
        in_specs=[
            pl.BlockSpec((B, _H), lambda i: (i, 0)),
            pl.BlockSpec((B, _H), lambda i: (i, 0)),
            pl.BlockSpec((B, _D), lambda i: (i, 0)),
            pl.BlockSpec((_D, _D), lambda i: (0, 0)),
            pl.BlockSpec((1, _D), lambda i: (0, 0)),
            pl.BlockSpec((_D, _D), lambda i: (0, 0)),
            pl.BlockSpec((1, _D), lambda i: (0, 0)),
        ],
        out_specs=[
            pl.BlockSpec((B, _D), lambda i: (i, 0)),
            pl.BlockSpec((B, _LANES), lambda i: (i, 0)),
            pl.BlockSpec((B, _LANES), lambda i: (i, 0)),
        ],
        out_shape=[
            jax.ShapeDtypeStruct((_NP, _D), jnp.float32),
            jax.ShapeDtypeStruct((_NP, _LANES), jnp.int32),
            jax.ShapeDtypeStruct((_NP, _LANES), jnp.int32),
        ],
    )
    return f(side_lo, side_hi, ego, Wg, bg, Wb, bb)


def kernel(user_emb, item_emb,
           W_gc_0, b_gc_0, W_bi_0, b_bi_0,
           W_gc_1, b_gc_1, W_bi_1, b_bi_1,
           W_gc_2, b_gc_2, W_bi_2, b_bi_2,
           edge_weight, edge_index):
    Ws = [(W_gc_0, b_gc_0, W_bi_0, b_bi_0),
          (W_gc_1, b_gc_1, W_bi_1, b_bi_1),
          (W_gc_2, b_gc_2, W_bi_2, b_bi_2)]
    ego = jnp.concatenate([user_emb, item_emb], axis=0)
    ego = jnp.pad(ego, ((0, _NP - _N), (0, 0)))
    src = edge_index[0].astype(jnp.int32)
    dst = edge_index[1].astype(jnp.int32)
    w = edge_weight.astype(jnp.float32)
    pad = _E_PAD - _E
    src2 = jnp.pad(src, (0, pad)).reshape(_ROWS2D, _MICRO)
    dst2 = jnp.pad(dst, (0, pad)).reshape(_ROWS2D, _MICRO)
    w2 = jnp.pad(w, (0, pad)).reshape(_ROWS2D, _MICRO)
    zeros = jnp.zeros((_NP, _H), jnp.float32)
    tlo = _pack_half(ego[:, 0:16], ego[:, 16:32])
    thi = _pack_half(ego[:, 32:48], ego[:, 48:64])
    outs = [ego[:_N]]
    for (Wg, bg, Wb, bb) in Ws:
        slo, shi = _sc_side(tlo, thi, src2, dst2, w2, zeros)
        ego, tlo, thi = _tc_dense(slo, shi, ego, Wg, bg, Wb, bb)
        outs.append(ego[:_N])
    return jnp.concatenate(outs, axis=1)
